# baseline (speedup 1.0000x reference)
import functools

import jax
import jax.numpy as jnp
from jax import lax
from jax.experimental import pallas as pl
from jax.experimental.pallas import tpu as pltpu
from jax.experimental.pallas import tpu_sc as plsc

RADIUS = 0.2
K = 16
N = 1024
QB = 256
CH = 640
CHP = 384
NW = 32
GQ = 8

_T0 = (((0,), (0,)), ((), ()))


def _to_bf16_bits_hi(x):
    u = lax.bitcast_convert_type(x, jnp.int32)
    r = u + 0x7FFF + lax.shift_right_logical(u, 16) % 2
    return lax.bitwise_and(r, jnp.int32(-65536))


def _stage1_body(h2_ref, c2_ref, p1_ref, p2_ref, p2t_ref,
                 wst_ref, woldst_ref, wd4t_ref, wdoldt_ref,
                 g_ref, idx_ref, tri_ref):
    q = pl.program_id(0)

    @pl.when(q == 0)
    def _build_tri():
        rows = lax.broadcasted_iota(jnp.int32, (N, N), 0)
        cols = lax.broadcasted_iota(jnp.int32, (N, N), 1)
        tri_ref[...] = jnp.where(rows <= cols, 1.0, 0.0).astype(jnp.bfloat16)

    p2 = p2_ref[...]
    g4 = lax.dot_general(h2_ref[...], wst_ref[...], _T0,
                         preferred_element_type=jnp.float32)
    g4 = g4 + jnp.dot(p2, wd4t_ref[...], preferred_element_type=jnp.float32)
    gold = lax.dot_general(c2_ref[...], woldst_ref[...], _T0,
                           preferred_element_type=jnp.float32)
    gold = gold + jnp.dot(p2, wdoldt_ref[...], preferred_element_type=jnp.float32)
    lo = jnp.concatenate([g4[:, 0:384]], axis=1)
    hi = jnp.concatenate([g4[:, 384:512], gold,
                          jnp.zeros((QB, 128), jnp.float32)], axis=1)
    word = lax.shift_right_logical(_to_bf16_bits_hi(lo), 16)
    g_ref[...] = lax.bitwise_or(word, _to_bf16_bits_hi(hi))

    p1 = p1_ref[...]
    p2t = p2t_ref[...]
    d2 = jnp.zeros((QB, N), jnp.float32)
    for c in range(3):
        diff = p1[:, c:c + 1] - p2t[c:c + 1, :]
        d2 = d2 + diff * diff
    mask = d2 < jnp.float32(RADIUS * RADIUS)
    maskbf = jnp.where(mask, 1.0, 0.0).astype(jnp.bfloat16)
    cum = jnp.dot(maskbf, tri_ref[...], preferred_element_type=jnp.float32)
    count = jnp.sum(jnp.where(mask, 1.0, 0.0), axis=1, keepdims=True)
    s = jnp.where(mask, cum, 0.0)
    jcol = lax.broadcasted_iota(jnp.int32, (QB, N), 1).astype(jnp.float32)
    jcol2 = jcol * 2048.0
    lo_cols, hi_cols = [], []
    for k in range(1, K // 2 + 1):
        v = jnp.where(s == k, jcol, 0.0) + jnp.where(s == k + 8, jcol2, 0.0)
        t = jnp.sum(v, axis=1, keepdims=True)
        hi = jnp.floor(t * (1.0 / 2048.0))
        lo_cols.append(t - 2048.0 * hi)
        hi_cols.append(hi)
    idx1 = lo_cols[0]
    cols = [jnp.where(count >= k + 1, c, idx1)
            for k, c in enumerate(lo_cols + hi_cols)]
    idx = jnp.concatenate(cols, axis=1)
    idx_ref[...] = idx.astype(jnp.int32)


def _sc_gather_max_body(g_hbm, idx_hbm, out_hbm, idx_v, rows0, rows1,
                        outa, outb, sem0, sem1):
    qw = N // NW
    nchunk = qw // GQ
    wid = lax.axis_index("s") * 2 + lax.axis_index("c")
    qbase = wid * qw
    pltpu.sync_copy(idx_hbm.at[pl.ds(qbase * K, qw * K)], idx_v)

    def _fire(chunk, rows, sem):
        pltpu.async_copy(g_hbm.at[idx_v.at[pl.ds(chunk * GQ * K, GQ * K)]],
                         rows, sem)

    def _drain(rows, sem):
        pltpu.make_async_copy(g_hbm.at[pl.ds(0, GQ * K)], rows, sem).wait()

    def _reduce(rows, out, chunk):
        rows_bf = rows.bitcast(jnp.bfloat16)
        out_bf = out.bitcast(jnp.bfloat16)

        def one_query(qi, carry):
            for c in range(CHP // 16):
                acc = rows_bf[pl.ds(2 * (qi * K), 2), pl.ds(c * 16, 16)]
                for r in range(1, K):
                    acc = jnp.maximum(
                        acc,
                        rows_bf[pl.ds(2 * (qi * K + r), 2), pl.ds(c * 16, 16)])
                out_bf[pl.ds(2 * qi, 2), pl.ds(c * 16, 16)] = acc
            return carry

        lax.fori_loop(0, GQ, one_query, 0)
        pltpu.sync_copy(out, out_hbm.at[pl.ds(qbase + chunk * GQ, GQ)])

    _fire(0, rows0, sem0)

    def body(i, carry):
        c0 = 2 * i
        c1 = c0 + 1
        _fire(c1, rows1, sem1)
        _drain(rows0, sem0)
        _reduce(rows0, outa, c0)

        @pl.when(i < nchunk // 2 - 1)
        def _prefetch():
            _fire(c0 + 2, rows0, sem0)

        _drain(rows1, sem1)
        _reduce(rows1, outb, c1)
        return carry

    lax.fori_loop(0, nchunk // 2, body, 0)


def _stage3_body(m_ref, x1_ref, p1_ref, wxt_ref, wd4t_ref, wdoldt_ref,
                 b4_ref, bold_ref, h1_ref, c1_ref):
    mw = m_ref[...]
    a_lo = lax.bitcast_convert_type(lax.shift_left(mw, 16), jnp.float32)
    a_hi = lax.bitcast_convert_type(
        lax.bitwise_and(mw, jnp.int32(-65536)), jnp.float32)
    p1 = p1_ref[...]
    c4 = lax.dot_general(x1_ref[...], wxt_ref[...], _T0,
                         preferred_element_type=jnp.float32)
    c4 = c4 - jnp.dot(p1, wd4t_ref[...], preferred_element_type=jnp.float32)
    c4 = c4 + b4_ref[...]
    cold = bold_ref[...] - jnp.dot(p1, wdoldt_ref[...],
                                   preferred_element_type=jnp.float32)
    lo = jax.nn.relu(a_lo + c4[:, 0:384])
    hi = jax.nn.relu(a_hi[:, 0:256] +
                     jnp.concatenate([c4[:, 384:512], cold], axis=1))
    gi = jax.nn.sigmoid(lo[:, 0:128])
    gf = jax.nn.sigmoid(lo[:, 128:256])
    go = jax.nn.sigmoid(lo[:, 256:384])
    cn = jnp.tanh(hi[:, 0:128])
    co = hi[:, 128:256]
    c1 = gf * co + gi * cn
    h1_ref[...] = jnp.transpose(go * jnp.tanh(c1))
    c1_ref[...] = jnp.transpose(c1)


def _stage1_call(b, h2, c2, p1, p2, p2t, wst, woldst, wd4t, wdoldt):
    return pl.pallas_call(
        _stage1_body,
        grid=(N // QB,),
        in_specs=[
            pl.BlockSpec((None, 128, QB), lambda q, b=b: (b, 0, q)),
            pl.BlockSpec((None, 128, QB), lambda q, b=b: (b, 0, q)),
            pl.BlockSpec((None, QB, 3), lambda q, b=b: (b, q, 0)),
            pl.BlockSpec((None, QB, 3), lambda q, b=b: (b, q, 0)),
            pl.BlockSpec((None, 3, N), lambda q, b=b: (b, 0, 0)),
            pl.BlockSpec((128, 512), lambda q: (0, 0)),
            pl.BlockSpec((128, 128), lambda q: (0, 0)),
            pl.BlockSpec((3, 512), lambda q: (0, 0)),
            pl.BlockSpec((3, 128), lambda q: (0, 0)),
        ],
        out_specs=[
            pl.BlockSpec((QB, CHP), lambda q: (q, 0)),
            pl.BlockSpec((QB, K), lambda q: (q, 0)),
        ],
        out_shape=[
            jax.ShapeDtypeStruct((N, CHP), jnp.int32),
            jax.ShapeDtypeStruct((N, K), jnp.int32),
        ],
        scratch_shapes=[pltpu.VMEM((N, N), jnp.bfloat16)],
    )(h2, c2, p1, p2, p2t, wst, woldst, wd4t, wdoldt)


def _stage3_call(b, m_words, x1, p1, wxt, wd4t, wdoldt, b4, bold2):
    return pl.pallas_call(
        _stage3_body,
        grid=(N // QB,),
        in_specs=[
            pl.BlockSpec((QB, CHP), lambda q: (q, 0)),
            pl.BlockSpec((None, 128, QB), lambda q, b=b: (b, 0, q)),
            pl.BlockSpec((None, QB, 3), lambda q, b=b: (b, q, 0)),
            pl.BlockSpec((128, 512), lambda q: (0, 0)),
            pl.BlockSpec((3, 512), lambda q: (0, 0)),
            pl.BlockSpec((3, 128), lambda q: (0, 0)),
            pl.BlockSpec((1, 512), lambda q: (0, 0)),
            pl.BlockSpec((1, 128), lambda q: (0, 0)),
        ],
        out_specs=[
            pl.BlockSpec((128, QB), lambda q: (0, q)),
            pl.BlockSpec((128, QB), lambda q: (0, q)),
        ],
        out_shape=[
            jax.ShapeDtypeStruct((128, N), jnp.float32),
            jax.ShapeDtypeStruct((128, N), jnp.float32),
        ],
    )(m_words, x1, p1, wxt, wd4t, wdoldt, b4, bold2)


@jax.jit
def kernel(P1, X1, P2, H2, C2, Wi, bi, Wf, bf, Wo, bo, Wn, bn_, Wold, bold):
    B = P1.shape[0]
    W_ST = jnp.concatenate([Wi[:, :128], Wf[:, :128], Wo[:, :128], Wn[:, :128]], 0).T
    W_XT = jnp.concatenate([Wi[:, 128:256], Wf[:, 128:256], Wo[:, 128:256], Wn[:, 128:256]], 0).T
    W_D4T = jnp.concatenate([Wi[:, 256:], Wf[:, 256:], Wo[:, 256:], Wn[:, 256:]], 0).T
    WoldST = Wold[:, :128].T
    W_DoldT = Wold[:, 128:].T
    b4 = jnp.concatenate([bi, bf, bo, bn_], 0)[None, :]
    bold2 = bold[None, :]
    P2T = jnp.transpose(P2, (0, 2, 1))

    qw = N // NW
    sc = pl.kernel(
        _sc_gather_max_body,
        out_type=jax.ShapeDtypeStruct((N, CHP), jnp.int32),
        mesh=plsc.VectorSubcoreMesh(core_axis_name="c", subcore_axis_name="s"),
        scratch_types=[
            pltpu.VMEM((qw * K,), jnp.int32),
            pltpu.VMEM((GQ * K, CHP), jnp.int32),
            pltpu.VMEM((GQ * K, CHP), jnp.int32),
            pltpu.VMEM((GQ, CHP), jnp.int32),
            pltpu.VMEM((GQ, CHP), jnp.int32),
            pltpu.SemaphoreType.DMA,
            pltpu.SemaphoreType.DMA,
        ],
    )

    gs, idxs, h1s, c1s = [], [], [], []
    for b in range(B):
        g_w, idx = _stage1_call(b, H2, C2, P1, P2, P2T,
                                W_ST, WoldST, W_D4T, W_DoldT)
        gs.append(g_w)
        idxs.append(idx)
    for b in range(B):
        m_words = sc(gs[b], idxs[b].reshape(N * K))
        h1, c1 = _stage3_call(b, m_words, X1, P1, W_XT, W_D4T, W_DoldT,
                              b4, bold2)
        h1s.append(h1)
        c1s.append(c1)
    H1 = jnp.concatenate([h[None] for h in h1s], axis=0)
    C1 = jnp.concatenate([c[None] for c in c1s], axis=0)
    return (P1, H1, C1)


# speedup vs baseline: 1.0068x; 1.0068x over previous
import functools

import jax
import jax.numpy as jnp
from jax import lax
from jax.experimental import pallas as pl
from jax.experimental.pallas import tpu as pltpu
from jax.experimental.pallas import tpu_sc as plsc

RADIUS = 0.2
K = 16
N = 1024
QB = 256
CH = 640
CHP = 384
NW = 32
GQ = 8

_T0 = (((0,), (0,)), ((), ()))


def _to_bf16_bits_hi(x):
    u = lax.bitcast_convert_type(x, jnp.int32)
    r = u + 0x7FFF + lax.shift_right_logical(u, 16) % 2
    return lax.bitwise_and(r, jnp.int32(-65536))


def _stage1_body(h2_ref, c2_ref, p1_ref, p2_ref, p2t_ref,
                 wst_ref, woldst_ref, wd4t_ref, wdoldt_ref,
                 g_ref, idx_ref, tri_ref):
    q = pl.program_id(0)

    @pl.when(q == 0)
    def _build_tri():
        rows = lax.broadcasted_iota(jnp.int32, (N, N), 0)
        cols = lax.broadcasted_iota(jnp.int32, (N, N), 1)
        tri_ref[...] = jnp.where(rows <= cols, 1.0, 0.0).astype(jnp.bfloat16)

    p2 = p2_ref[...]
    g4 = lax.dot_general(h2_ref[...], wst_ref[...], _T0,
                         preferred_element_type=jnp.float32)
    g4 = g4 + jnp.dot(p2, wd4t_ref[...], preferred_element_type=jnp.float32)
    gold = lax.dot_general(c2_ref[...], woldst_ref[...], _T0,
                           preferred_element_type=jnp.float32)
    gold = gold + jnp.dot(p2, wdoldt_ref[...], preferred_element_type=jnp.float32)
    lo = jnp.concatenate([g4[:, 0:384]], axis=1)
    hi = jnp.concatenate([g4[:, 384:512], gold,
                          jnp.zeros((QB, 128), jnp.float32)], axis=1)
    word = lax.shift_right_logical(_to_bf16_bits_hi(lo), 16)
    g_ref[...] = lax.bitwise_or(word, _to_bf16_bits_hi(hi))

    p1 = p1_ref[...]
    p2t = p2t_ref[...]
    d2 = jnp.zeros((QB, N), jnp.float32)
    for c in range(3):
        diff = p1[:, c:c + 1] - p2t[c:c + 1, :]
        d2 = d2 + diff * diff
    mask = d2 < jnp.float32(RADIUS * RADIUS)
    maskbf = jnp.where(mask, 1.0, 0.0).astype(jnp.bfloat16)
    cum = jnp.dot(maskbf, tri_ref[...], preferred_element_type=jnp.float32)
    count = jnp.sum(jnp.where(mask, 1.0, 0.0), axis=1, keepdims=True)
    s = jnp.where(mask, cum, 0.0)
    jcol = lax.broadcasted_iota(jnp.int32, (QB, N), 1).astype(jnp.float32)
    cols = []
    idx1 = None
    for k in range(1, K + 1):
        idxk = jnp.sum(jnp.where(s == k, jcol, 0.0), axis=1, keepdims=True)
        if idx1 is None:
            idx1 = idxk
        cols.append(jnp.where(count >= k, idxk, idx1))
    idx = jnp.concatenate(cols, axis=1)
    idx_ref[...] = idx.astype(jnp.int32)


def _sc_gather_max_body(g_hbm, idx_hbm, out_hbm, idx_v, rows0, rows1,
                        outa, outb, sem0, sem1):
    qw = N // NW
    nchunk = qw // GQ
    wid = lax.axis_index("s") * 2 + lax.axis_index("c")
    qbase = wid * qw
    pltpu.sync_copy(idx_hbm.at[pl.ds(qbase * K, qw * K)], idx_v)

    def _fire(chunk, rows, sem):
        pltpu.async_copy(g_hbm.at[idx_v.at[pl.ds(chunk * GQ * K, GQ * K)]],
                         rows, sem)

    def _drain(rows, sem):
        pltpu.make_async_copy(g_hbm.at[pl.ds(0, GQ * K)], rows, sem).wait()

    def _reduce(rows, out, chunk):
        rows_bf = rows.bitcast(jnp.bfloat16)
        out_bf = out.bitcast(jnp.bfloat16)

        def one_query(qi, carry):
            for c in range(CHP // 16):
                acc = rows_bf[pl.ds(2 * (qi * K), 2), pl.ds(c * 16, 16)]
                for r in range(1, K):
                    acc = jnp.maximum(
                        acc,
                        rows_bf[pl.ds(2 * (qi * K + r), 2), pl.ds(c * 16, 16)])
                out_bf[pl.ds(2 * qi, 2), pl.ds(c * 16, 16)] = acc
            return carry

        lax.fori_loop(0, GQ, one_query, 0)
        pltpu.sync_copy(out, out_hbm.at[pl.ds(qbase + chunk * GQ, GQ)])

    _fire(0, rows0, sem0)

    def body(i, carry):
        c0 = 2 * i
        c1 = c0 + 1
        _fire(c1, rows1, sem1)
        _drain(rows0, sem0)
        _reduce(rows0, outa, c0)

        @pl.when(i < nchunk // 2 - 1)
        def _prefetch():
            _fire(c0 + 2, rows0, sem0)

        _drain(rows1, sem1)
        _reduce(rows1, outb, c1)
        return carry

    lax.fori_loop(0, nchunk // 2, body, 0)


def _stage3_body(m_ref, x1_ref, p1_ref, wxt_ref, wd4t_ref, wdoldt_ref,
                 b4_ref, bold_ref, h1_ref, c1_ref):
    mw = m_ref[...]
    a_lo = lax.bitcast_convert_type(lax.shift_left(mw, 16), jnp.float32)
    a_hi = lax.bitcast_convert_type(
        lax.bitwise_and(mw, jnp.int32(-65536)), jnp.float32)
    p1 = p1_ref[...]
    c4 = lax.dot_general(x1_ref[...], wxt_ref[...], _T0,
                         preferred_element_type=jnp.float32)
    c4 = c4 - jnp.dot(p1, wd4t_ref[...], preferred_element_type=jnp.float32)
    c4 = c4 + b4_ref[...]
    cold = bold_ref[...] - jnp.dot(p1, wdoldt_ref[...],
                                   preferred_element_type=jnp.float32)
    lo = jax.nn.relu(a_lo + c4[:, 0:384])
    hi = jax.nn.relu(a_hi[:, 0:256] +
                     jnp.concatenate([c4[:, 384:512], cold], axis=1))
    gi = jax.nn.sigmoid(lo[:, 0:128])
    gf = jax.nn.sigmoid(lo[:, 128:256])
    go = jax.nn.sigmoid(lo[:, 256:384])
    cn = jnp.tanh(hi[:, 0:128])
    co = hi[:, 128:256]
    c1 = gf * co + gi * cn
    h1_ref[...] = jnp.transpose(go * jnp.tanh(c1))
    c1_ref[...] = jnp.transpose(c1)


def _stage1_call(b, h2, c2, p1, p2, p2t, wst, woldst, wd4t, wdoldt):
    return pl.pallas_call(
        _stage1_body,
        grid=(N // QB,),
        in_specs=[
            pl.BlockSpec((None, 128, QB), lambda q, b=b: (b, 0, q)),
            pl.BlockSpec((None, 128, QB), lambda q, b=b: (b, 0, q)),
            pl.BlockSpec((None, QB, 3), lambda q, b=b: (b, q, 0)),
            pl.BlockSpec((None, QB, 3), lambda q, b=b: (b, q, 0)),
            pl.BlockSpec((None, 3, N), lambda q, b=b: (b, 0, 0)),
            pl.BlockSpec((128, 512), lambda q: (0, 0)),
            pl.BlockSpec((128, 128), lambda q: (0, 0)),
            pl.BlockSpec((3, 512), lambda q: (0, 0)),
            pl.BlockSpec((3, 128), lambda q: (0, 0)),
        ],
        out_specs=[
            pl.BlockSpec((QB, CHP), lambda q: (q, 0)),
            pl.BlockSpec((QB, K), lambda q: (q, 0)),
        ],
        out_shape=[
            jax.ShapeDtypeStruct((N, CHP), jnp.int32),
            jax.ShapeDtypeStruct((N, K), jnp.int32),
        ],
        scratch_shapes=[pltpu.VMEM((N, N), jnp.bfloat16)],
    )(h2, c2, p1, p2, p2t, wst, woldst, wd4t, wdoldt)


def _stage3_call(b, m_words, x1, p1, wxt, wd4t, wdoldt, b4, bold2):
    return pl.pallas_call(
        _stage3_body,
        grid=(N // QB,),
        in_specs=[
            pl.BlockSpec((QB, CHP), lambda q: (q, 0)),
            pl.BlockSpec((None, 128, QB), lambda q, b=b: (b, 0, q)),
            pl.BlockSpec((None, QB, 3), lambda q, b=b: (b, q, 0)),
            pl.BlockSpec((128, 512), lambda q: (0, 0)),
            pl.BlockSpec((3, 512), lambda q: (0, 0)),
            pl.BlockSpec((3, 128), lambda q: (0, 0)),
            pl.BlockSpec((1, 512), lambda q: (0, 0)),
            pl.BlockSpec((1, 128), lambda q: (0, 0)),
        ],
        out_specs=[
            pl.BlockSpec((128, QB), lambda q: (0, q)),
            pl.BlockSpec((128, QB), lambda q: (0, q)),
        ],
        out_shape=[
            jax.ShapeDtypeStruct((128, N), jnp.float32),
            jax.ShapeDtypeStruct((128, N), jnp.float32),
        ],
    )(m_words, x1, p1, wxt, wd4t, wdoldt, b4, bold2)


@jax.jit
def kernel(P1, X1, P2, H2, C2, Wi, bi, Wf, bf, Wo, bo, Wn, bn_, Wold, bold):
    B = P1.shape[0]
    W_ST = jnp.concatenate([Wi[:, :128], Wf[:, :128], Wo[:, :128], Wn[:, :128]], 0).T
    W_XT = jnp.concatenate([Wi[:, 128:256], Wf[:, 128:256], Wo[:, 128:256], Wn[:, 128:256]], 0).T
    W_D4T = jnp.concatenate([Wi[:, 256:], Wf[:, 256:], Wo[:, 256:], Wn[:, 256:]], 0).T
    WoldST = Wold[:, :128].T
    W_DoldT = Wold[:, 128:].T
    b4 = jnp.concatenate([bi, bf, bo, bn_], 0)[None, :]
    bold2 = bold[None, :]
    P2T = jnp.transpose(P2, (0, 2, 1))

    qw = N // NW
    sc = pl.kernel(
        _sc_gather_max_body,
        out_type=jax.ShapeDtypeStruct((N, CHP), jnp.int32),
        mesh=plsc.VectorSubcoreMesh(core_axis_name="c", subcore_axis_name="s"),
        scratch_types=[
            pltpu.VMEM((qw * K,), jnp.int32),
            pltpu.VMEM((GQ * K, CHP), jnp.int32),
            pltpu.VMEM((GQ * K, CHP), jnp.int32),
            pltpu.VMEM((GQ, CHP), jnp.int32),
            pltpu.VMEM((GQ, CHP), jnp.int32),
            pltpu.SemaphoreType.DMA,
            pltpu.SemaphoreType.DMA,
        ],
    )

    gs, idxs, h1s, c1s = [], [], [], []
    for b in range(B):
        g_w, idx = _stage1_call(b, H2, C2, P1, P2, P2T,
                                W_ST, WoldST, W_D4T, W_DoldT)
        gs.append(g_w)
        idxs.append(idx)
    for b in range(B):
        m_words = sc(gs[b], idxs[b].reshape(N * K))
        h1, c1 = _stage3_call(b, m_words, X1, P1, W_XT, W_D4T, W_DoldT,
                              b4, bold2)
        h1s.append(h1)
        c1s.append(c1)
    H1 = jnp.concatenate([h[None] for h in h1s], axis=0)
    C1 = jnp.concatenate([c[None] for c in c1s], axis=0)
    return (P1, H1, C1)
